# baseline (speedup 1.0000x reference)
import jax
import jax.numpy as jnp
from jax.experimental import pallas as pl
from jax.experimental.pallas import tpu as pltpu

_N = 128
_M = _N * (_N + 1) // 2
_BLK = 8


def _body(x_ref, o_ref):
    x = x_ref[...]
    rows = []
    for r in range(_N):
        s = 127 * r - (r * (r - 1)) // 2
        rows.append(x[:, s:s + _N])
    t = jnp.stack(rows, axis=1)
    tt = jnp.swapaxes(t, 1, 2)
    col = jax.lax.broadcasted_iota(jnp.int32, (_N, _N), 1)
    row = jax.lax.broadcasted_iota(jnp.int32, (_N, _N), 0)
    upper = (col >= row)[None, :, :]
    o_ref[...] = jnp.where(upper, t, tt)


def kernel(input):
    b = input.shape[0]
    assert input.shape[1] == _M and b % _BLK == 0
    return pl.pallas_call(
        _body,
        grid=(b // _BLK,),
        in_specs=[pl.BlockSpec((_BLK, _M), lambda i: (i, 0))],
        out_specs=pl.BlockSpec((_BLK, _N, _N), lambda i: (i, 0, 0)),
        out_shape=jax.ShapeDtypeStruct((b, _N, _N), input.dtype),
    )(input)


# speedup vs baseline: 1.2423x; 1.2423x over previous
import functools

import jax
import jax.numpy as jnp
from jax import lax
from jax.experimental import pallas as pl
from jax.experimental.pallas import tpu as pltpu
from jax.experimental.pallas import tpu_sc as plsc

_N = 128
_M = _N * (_N + 1) // 2
_NC = 2
_NS = 16
_NW = _NC * _NS
_L = 16
_NCHUNK = _N // _L


def _sc_body(x_hbm, o_hbm, xv, ov, nper):
    wid = lax.axis_index("s") * _NC + lax.axis_index("c")
    jvs = [lax.iota(jnp.int32, 16) + _L * k for k in range(_NCHUNK)]
    svs = [127 * j - ((j * (j - 1)) >> 1) for j in jvs]

    def sample_body(i, carry):
        b = wid * nper + i
        pltpu.sync_copy(x_hbm.at[b], xv)

        def row_body(r, carry2):
            base = 127 * r - ((r * (r - 1)) >> 1)
            for k in range(_NCHUNK):
                idx = jnp.where(jvs[k] < r, svs[k] + r, base + jvs[k])
                ov[pl.ds(r * _N + _L * k, _L)] = plsc.load_gather(xv, [idx])
            return carry2

        lax.fori_loop(0, _N, row_body, 0, unroll=False)
        pltpu.sync_copy(ov, o_hbm.at[b])
        return carry

    lax.fori_loop(0, nper, sample_body, 0, unroll=False)


def kernel(input):
    b = input.shape[0]
    assert input.shape[1] == _M and b % _NW == 0
    nper = b // _NW
    mesh = plsc.VectorSubcoreMesh(
        core_axis_name="c", subcore_axis_name="s",
        num_cores=_NC, num_subcores=_NS)
    out = pl.kernel(
        functools.partial(_sc_body, nper=nper),
        out_type=jax.ShapeDtypeStruct((b, _N * _N), input.dtype),
        mesh=mesh,
        scratch_types=[
            pltpu.VMEM((_M,), jnp.float32),
            pltpu.VMEM((_N * _N,), jnp.float32),
        ],
        compiler_params=pltpu.CompilerParams(needs_layout_passes=False),
    )(input)
    return out.reshape(b, _N, _N)


# speedup vs baseline: 1.5848x; 1.2757x over previous
import functools

import jax
import jax.numpy as jnp
from jax import lax
from jax.experimental import pallas as pl
from jax.experimental.pallas import tpu as pltpu
from jax.experimental.pallas import tpu_sc as plsc

_N = 128
_M = _N * (_N + 1) // 2
_NC = 2
_NS = 16
_NW = _NC * _NS
_L = 16
_NCHUNK = _N // _L


def _sc_body(x_hbm, o_hbm, xv0, xv1, ov0, ov1, isem0, isem1, osem0, osem1,
             nper):
    wid = lax.axis_index("s") * _NC + lax.axis_index("c")
    b0 = wid * nper
    xvs = (xv0, xv1)
    ovs = (ov0, ov1)
    isems = (isem0, isem1)
    osems = (osem0, osem1)
    jvs = [lax.iota(jnp.int32, _L) + _L * k for k in range(_NCHUNK)]
    svs = [127 * j - ((j * (j - 1)) >> 1) for j in jvs]

    def gather_sample(par):
        def row_body(r, carry):
            base = 127 * r - ((r * (r - 1)) >> 1)
            for k in range(_NCHUNK):
                idx = jnp.where(jvs[k] < r, svs[k] + r, base + jvs[k])
                ovs[par][pl.ds(r * _N + _L * k, _L)] = plsc.load_gather(
                    xvs[par], [idx])
            return carry

        lax.fori_loop(0, _N, row_body, 0, unroll=4)

    pltpu.async_copy(x_hbm.at[b0], xvs[0], isems[0])
    pltpu.async_copy(x_hbm.at[b0 + 1], xvs[1], isems[1])

    def pair_body(p, carry):
        for par in range(2):
            i = 2 * p + par
            b = b0 + i
            pltpu.make_async_copy(x_hbm.at[b], xvs[par], isems[par]).wait()

            @pl.when(p >= 1)
            def _wait_out():
                pltpu.make_async_copy(ovs[par], o_hbm.at[b - 2],
                                      osems[par]).wait()

            gather_sample(par)
            pltpu.async_copy(ovs[par], o_hbm.at[b], osems[par])

            @pl.when(p < nper // 2 - 1)
            def _next_in():
                pltpu.async_copy(x_hbm.at[b + 2], xvs[par], isems[par])

        return carry

    lax.fori_loop(0, nper // 2, pair_body, 0, unroll=False)
    pltpu.make_async_copy(ovs[0], o_hbm.at[b0], osems[0]).wait()
    pltpu.make_async_copy(ovs[1], o_hbm.at[b0], osems[1]).wait()


def kernel(input):
    b = input.shape[0]
    assert input.shape[1] == _M and b % (2 * _NW) == 0
    nper = b // _NW
    mesh = plsc.VectorSubcoreMesh(
        core_axis_name="c", subcore_axis_name="s",
        num_cores=_NC, num_subcores=_NS)
    out = pl.kernel(
        functools.partial(_sc_body, nper=nper),
        out_type=jax.ShapeDtypeStruct((b, _N * _N), input.dtype),
        mesh=mesh,
        scratch_types=[
            pltpu.VMEM((_M,), jnp.float32),
            pltpu.VMEM((_M,), jnp.float32),
            pltpu.VMEM((_N * _N,), jnp.float32),
            pltpu.VMEM((_N * _N,), jnp.float32),
            pltpu.SemaphoreType.DMA,
            pltpu.SemaphoreType.DMA,
            pltpu.SemaphoreType.DMA,
            pltpu.SemaphoreType.DMA,
        ],
        compiler_params=pltpu.CompilerParams(needs_layout_passes=False),
    )(input)
    return out.reshape(b, _N, _N)
